# baseline (speedup 1.0000x reference)
import functools

import jax
import jax.numpy as jnp
from jax import lax
from jax.experimental import pallas as pl
from jax.experimental.pallas import tpu as pltpu
from jax.experimental.pallas import tpu_sc as plsc

N = 10000
NP = 10240
E = 320000
D = 128
NC = 2
NS = 16
NW = NC * NS
EPW = E // NW
CH = 64
CHD = 128
RPT = N // NS
PPT = NP // NS

_MESH = plsc.VectorSubcoreMesh(core_axis_name="c", subcore_axis_name="s")


def _worker_id():
    cid = lax.axis_index("c")
    sid = lax.axis_index("s")
    return cid, sid, sid * NC + cid


NCHUNK = E // CH
NFULLR = NCHUNK // NW
NEXTRA = NCHUNK - NFULLR * NW
NCHUNKD = E // CHD
NFULLT = NCHUNKD // NS
NEXTRT = NCHUNKD - NFULLT * NS


def _rsqrt16(x):
    xi = lax.bitcast_convert_type(x, jnp.int32)
    yi = jnp.full((16,), 0x5F3759DF, jnp.int32) - lax.shift_right_logical(
        xi, jnp.full((16,), 1, jnp.int32))
    y = lax.bitcast_convert_type(yi, jnp.float32)
    for _ in range(3):
        y = y * (1.5 - 0.5 * x * y * y)
    return y


@functools.partial(
    pl.kernel,
    out_type=jax.ShapeDtypeStruct((NCHUNK, 3, CH), jnp.int32),
    mesh=_MESH,
    scratch_types=[
        pltpu.VMEM((3, CH), jnp.int32),
        pltpu.VMEM((CH,), jnp.float32),
        pltpu.VMEM((CH,), jnp.float32),
        pltpu.VMEM((CH,), jnp.float32),
        pltpu.VMEM((CHD,), jnp.int32),
        pltpu.VMEM((CHD,), jnp.float32),
        pltpu.VMEM((PPT,), jnp.float32),
        pltpu.VMEM_SHARED((NP,), jnp.float32),
        pltpu.VMEM_SHARED((NP,), jnp.float32),
        pltpu.SemaphoreType.DMA,
        pltpu.SemaphoreType.DMA,
    ],
)
def _prep_kernel(row_hbm, col_hbm, w_hbm, zeros_hbm, out_hbm,
                 ebuf, w_v, dr_v, dc_v, didx_v, dw_v, dbuf,
                 deg_sh, dinv_sh, sem_r, sem_c):
    _, sid, wid = _worker_id()

    pltpu.sync_copy(zeros_hbm.at[pl.ds(sid * PPT, PPT)],
                    deg_sh.at[pl.ds(sid * PPT, PPT)])
    plsc.subcore_barrier()

    def deg_chunk(c):
        off = pl.multiple_of(c * CHD, 8)
        pltpu.sync_copy(row_hbm.at[pl.ds(off, CHD)], didx_v)
        pltpu.sync_copy(w_hbm.at[pl.ds(off, CHD)], dw_v)
        pltpu.sync_copy(dw_v, deg_sh.at[didx_v], add=True)

    @pl.loop(0, NFULLT)
    def _deg(g):
        deg_chunk(g * NS + sid)

    @pl.when(sid < NEXTRT)
    def _deg_extra():
        deg_chunk(NFULLT * NS + sid)

    plsc.subcore_barrier()

    pltpu.sync_copy(deg_sh.at[pl.ds(sid * PPT, PPT)], dbuf)
    for j in range(PPT // 16):
        sl = pl.ds(j * 16, 16)
        d = dbuf[sl]
        pos = d > 0.0
        safe = jnp.where(pos, d, 1.0)
        dbuf[sl] = jnp.where(pos, _rsqrt16(safe), 0.0)
    pltpu.sync_copy(dbuf, dinv_sh.at[pl.ds(sid * PPT, PPT)])
    plsc.subcore_barrier()

    def norm_chunk(c):
        off = pl.multiple_of(c * CH, 8)
        pltpu.sync_copy(row_hbm.at[pl.ds(off, CH)], ebuf.at[0])
        pltpu.sync_copy(col_hbm.at[pl.ds(off, CH)], ebuf.at[1])
        pltpu.sync_copy(w_hbm.at[pl.ds(off, CH)], w_v)
        a = pltpu.async_copy(dinv_sh.at[ebuf.at[0]], dr_v, sem_r)
        b = pltpu.async_copy(dinv_sh.at[ebuf.at[1]], dc_v, sem_c)
        a.wait()
        b.wait()
        for j in range(CH // 16):
            sl = pl.ds(j * 16, 16)
            ebuf[2, sl] = lax.bitcast_convert_type(
                -(dr_v[sl] * w_v[sl] * dc_v[sl]), jnp.int32)
        pltpu.sync_copy(ebuf, out_hbm.at[c])

    @pl.loop(0, NFULLR)
    def _chunks(g):
        norm_chunk(g * NW + wid)

    @pl.when(wid < NEXTRA)
    def _extra():
        norm_chunk(NFULLR * NW + wid)


NB = 4
NOUT = NFULLR // NB


@functools.partial(
    pl.kernel,
    out_type=jax.ShapeDtypeStruct((NC, NP, D), jnp.float32),
    mesh=_MESH,
    scratch_types=[
        [pltpu.VMEM((3, CH), jnp.int32) for _ in range(NB)],
        [pltpu.VMEM((CH,), jnp.int32) for _ in range(NB)],
        [pltpu.VMEM((CH,), jnp.int32) for _ in range(NB)],
        [pltpu.VMEM((CH,), jnp.int32) for _ in range(NB)],
        [pltpu.VMEM((CH, D), jnp.float32) for _ in range(NB)],
        pltpu.VMEM_SHARED((NP, D), jnp.float32),
        [pltpu.SemaphoreType.DMA for _ in range(NB)],
        [pltpu.SemaphoreType.DMA for _ in range(NB)],
        [pltpu.SemaphoreType.DMA for _ in range(NB)],
    ],
)
def _spmm_kernel(x_hbm, edata_hbm, zeros_hbm, out_hbm,
                 ebufs, rbufs, cbufs, nbufs, rowss, acc_sh,
                 sems_e, sems_g, sems_s):
    cid, sid, wid = _worker_id()
    pltpu.sync_copy(zeros_hbm.at[pl.ds(sid * PPT, PPT)],
                    acc_sh.at[pl.ds(sid * PPT, PPT)])
    plsc.subcore_barrier()

    def scale_rows(rows, nbuf):
        for g in range(CH // 16):
            nv = lax.bitcast_convert_type(nbuf[pl.ds(g * 16, 16)],
                                          jnp.float32)
            for j in range(16):
                e = g * 16 + j
                spl = lax.gather(
                    nv, jnp.full((16, 1), j, jnp.int32),
                    lax.GatherDimensionNumbers(
                        offset_dims=(), collapsed_slice_dims=(0,),
                        start_index_map=(0,)),
                    slice_sizes=(1,),
                    mode=lax.GatherScatterMode.PROMISE_IN_BOUNDS)
                for s in range(D // 16):
                    sl = pl.ds(s * 16, 16)
                    rows[e, sl] = rows[e, sl] * spl

    def unpack_ebuf(b):
        for r, dst in ((0, rbufs[b]), (1, cbufs[b]), (2, nbufs[b])):
            for s in range(CH // 16):
                sl = pl.ds(s * 16, 16)
                dst[sl] = ebufs[b][r, sl]

    def start_scatter(b):
        return pltpu.async_copy(rowss[b], acc_sh.at[cbufs[b]], sems_s[b],
                                add=True)

    def drain_scatter(b):
        pltpu.make_async_copy(rowss[b], acc_sh.at[cbufs[b]],
                              sems_s[b]).wait()

    for b in range(NB):
        pltpu.async_copy(edata_hbm.at[b * NW + wid], ebufs[b], sems_e[b])

    @pl.loop(0, NOUT)
    def _outer(g):
        dg = []
        for b in range(NB):
            @pl.when(g > 0)
            def _drain():
                drain_scatter(b)
            pltpu.make_async_copy(edata_hbm.at[0], ebufs[b], sems_e[b]).wait()
            unpack_ebuf(b)
            dg.append(pltpu.async_copy(x_hbm.at[rbufs[b]], rowss[b],
                                       sems_g[b]))

            @pl.when(g < NOUT - 1)
            def _prefetch():
                pltpu.async_copy(
                    edata_hbm.at[((g + 1) * NB + b) * NW + wid],
                    ebufs[b], sems_e[b])
        for b in range(NB):
            dg[b].wait()
            scale_rows(rowss[b], nbufs[b])
            start_scatter(b)

    for b in range(NB):
        drain_scatter(b)

    @pl.when(wid < NEXTRA)
    def _extra():
        c = NFULLR * NW + wid
        pltpu.async_copy(edata_hbm.at[c], ebufs[0], sems_e[0]).wait()
        unpack_ebuf(0)
        pltpu.async_copy(x_hbm.at[rbufs[0]], rowss[0], sems_g[0]).wait()
        scale_rows(rowss[0], nbufs[0])
        start_scatter(0).wait()

    plsc.subcore_barrier()
    pltpu.sync_copy(acc_sh.at[pl.ds(sid * PPT, PPT)],
                    out_hbm.at[cid, pl.ds(sid * PPT, PPT)])


def _combine_body(p_ref, out_ref):
    out_ref[...] = p_ref[0] + p_ref[1]


def _layer_body(h_ref, s1_ref, p2_ref, w0_ref, w1_ref, w2_ref, b_ref, out_ref):
    w0 = w0_ref[...] - w2_ref[...]
    w2 = 2.0 * w2_ref[...]
    t2 = p2_ref[0] + p2_ref[1]
    acc = jnp.dot(h_ref[...], w0, preferred_element_type=jnp.float32)
    acc += jnp.dot(s1_ref[...], w1_ref[...], preferred_element_type=jnp.float32)
    acc += jnp.dot(t2, w2, preferred_element_type=jnp.float32)
    acc += b_ref[...]
    out_ref[...] = 1.0 / (1.0 + jnp.exp(-acc))


_RB = 1024

_combine = pl.pallas_call(
    _combine_body,
    grid=(NP // _RB,),
    in_specs=[pl.BlockSpec((NC, _RB, D), lambda i: (0, i, 0))],
    out_specs=pl.BlockSpec((_RB, D), lambda i: (i, 0)),
    out_shape=jax.ShapeDtypeStruct((NP, D), jnp.float32),
)

_layer = pl.pallas_call(
    _layer_body,
    grid=(NP // _RB,),
    in_specs=[
        pl.BlockSpec((_RB, D), lambda i: (i, 0)),
        pl.BlockSpec((_RB, D), lambda i: (i, 0)),
        pl.BlockSpec((NC, _RB, D), lambda i: (0, i, 0)),
        pl.BlockSpec((D, D), lambda i: (0, 0)),
        pl.BlockSpec((D, D), lambda i: (0, 0)),
        pl.BlockSpec((D, D), lambda i: (0, 0)),
        pl.BlockSpec((1, D), lambda i: (0, 0)),
    ],
    out_specs=pl.BlockSpec((_RB, D), lambda i: (i, 0)),
    out_shape=jax.ShapeDtypeStruct((NP, D), jnp.float32),
)


def kernel(x, edge_index, edge_weight, W1, b1, W2, b2, W3, b3):
    row = edge_index[0]
    col = edge_index[1]
    zeros_np = jnp.zeros((NP,), jnp.float32)
    zeros_nd = jnp.zeros((NP, D), jnp.float32)

    edata = _prep_kernel(row, col, edge_weight, zeros_np)

    h = jnp.pad(x, ((0, NP - N), (0, 0)))
    for W, b in ((W1, b1), (W2, b2), (W3, b3)):
        p1 = _spmm_kernel(h, edata, zeros_nd)
        s1 = _combine(p1)
        p2 = _spmm_kernel(s1, edata, zeros_nd)
        h = _layer(h, s1, p2, W[0], W[1], W[2], b.reshape(1, D))
    return h[:N]


# speedup vs baseline: 1.0536x; 1.0536x over previous
import functools

import jax
import jax.numpy as jnp
from jax import lax
from jax.experimental import pallas as pl
from jax.experimental.pallas import tpu as pltpu
from jax.experimental.pallas import tpu_sc as plsc

N = 10000
NP = 10240
E = 320000
D = 128
NC = 2
NS = 16
NW = NC * NS
EPW = E // NW
CH = 128
CHD = 128
RPT = N // NS
PPT = NP // NS

_MESH = plsc.VectorSubcoreMesh(core_axis_name="c", subcore_axis_name="s")


def _worker_id():
    cid = lax.axis_index("c")
    sid = lax.axis_index("s")
    return cid, sid, sid * NC + cid


NCHUNK = E // CH
NFULLR = NCHUNK // NW
NEXTRA = NCHUNK - NFULLR * NW
NCHUNKD = E // CHD
NFULLT = NCHUNKD // NS
NEXTRT = NCHUNKD - NFULLT * NS


def _rsqrt16(x):
    xi = lax.bitcast_convert_type(x, jnp.int32)
    yi = jnp.full((16,), 0x5F3759DF, jnp.int32) - lax.shift_right_logical(
        xi, jnp.full((16,), 1, jnp.int32))
    y = lax.bitcast_convert_type(yi, jnp.float32)
    for _ in range(3):
        y = y * (1.5 - 0.5 * x * y * y)
    return y


@functools.partial(
    pl.kernel,
    out_type=jax.ShapeDtypeStruct((NCHUNK, 3, CH), jnp.int32),
    mesh=_MESH,
    scratch_types=[
        pltpu.VMEM((3, CH), jnp.int32),
        pltpu.VMEM((CH,), jnp.float32),
        pltpu.VMEM((CH,), jnp.float32),
        pltpu.VMEM((CH,), jnp.float32),
        pltpu.VMEM((CHD,), jnp.int32),
        pltpu.VMEM((CHD,), jnp.float32),
        pltpu.VMEM((PPT,), jnp.float32),
        pltpu.VMEM_SHARED((NP,), jnp.float32),
        pltpu.VMEM_SHARED((NP,), jnp.float32),
        pltpu.SemaphoreType.DMA,
        pltpu.SemaphoreType.DMA,
    ],
)
def _prep_kernel(row_hbm, col_hbm, w_hbm, zeros_hbm, out_hbm,
                 ebuf, w_v, dr_v, dc_v, didx_v, dw_v, dbuf,
                 deg_sh, dinv_sh, sem_r, sem_c):
    _, sid, wid = _worker_id()

    pltpu.sync_copy(zeros_hbm.at[pl.ds(sid * PPT, PPT)],
                    deg_sh.at[pl.ds(sid * PPT, PPT)])
    plsc.subcore_barrier()

    def deg_chunk(c):
        off = pl.multiple_of(c * CHD, 8)
        pltpu.sync_copy(row_hbm.at[pl.ds(off, CHD)], didx_v)
        pltpu.sync_copy(w_hbm.at[pl.ds(off, CHD)], dw_v)
        pltpu.sync_copy(dw_v, deg_sh.at[didx_v], add=True)

    @pl.loop(0, NFULLT)
    def _deg(g):
        deg_chunk(g * NS + sid)

    @pl.when(sid < NEXTRT)
    def _deg_extra():
        deg_chunk(NFULLT * NS + sid)

    plsc.subcore_barrier()

    pltpu.sync_copy(deg_sh.at[pl.ds(sid * PPT, PPT)], dbuf)
    for j in range(PPT // 16):
        sl = pl.ds(j * 16, 16)
        d = dbuf[sl]
        pos = d > 0.0
        safe = jnp.where(pos, d, 1.0)
        dbuf[sl] = jnp.where(pos, _rsqrt16(safe), 0.0)
    pltpu.sync_copy(dbuf, dinv_sh.at[pl.ds(sid * PPT, PPT)])
    plsc.subcore_barrier()

    def norm_chunk(c):
        off = pl.multiple_of(c * CH, 8)
        pltpu.sync_copy(row_hbm.at[pl.ds(off, CH)], ebuf.at[0])
        pltpu.sync_copy(col_hbm.at[pl.ds(off, CH)], ebuf.at[1])
        pltpu.sync_copy(w_hbm.at[pl.ds(off, CH)], w_v)
        a = pltpu.async_copy(dinv_sh.at[ebuf.at[0]], dr_v, sem_r)
        b = pltpu.async_copy(dinv_sh.at[ebuf.at[1]], dc_v, sem_c)
        a.wait()
        b.wait()
        for j in range(CH // 16):
            sl = pl.ds(j * 16, 16)
            ebuf[2, sl] = lax.bitcast_convert_type(
                -(dr_v[sl] * w_v[sl] * dc_v[sl]), jnp.int32)
        pltpu.sync_copy(ebuf, out_hbm.at[c])

    @pl.loop(0, NFULLR)
    def _chunks(g):
        norm_chunk(g * NW + wid)

    @pl.when(wid < NEXTRA)
    def _extra():
        norm_chunk(NFULLR * NW + wid)


NB = 3
NOUT = NFULLR // NB
RPT15 = 632


@functools.partial(
    pl.kernel,
    out_type=jax.ShapeDtypeStruct((NC, NP, D), jnp.float32),
    mesh=_MESH,
    scratch_types=[
        [pltpu.VMEM((3, CH), jnp.int32) for _ in range(NB)],
        [pltpu.VMEM((CH,), jnp.int32) for _ in range(NB)],
        [pltpu.VMEM((CH, D), jnp.float32) for _ in range(NB)],
        pltpu.VMEM_SHARED((N, D), jnp.float32),
        [pltpu.SemaphoreType.DMA for _ in range(NB)],
        [pltpu.SemaphoreType.DMA for _ in range(NB)],
        [pltpu.SemaphoreType.DMA for _ in range(NB)],
    ],
)
def _spmm_kernel(x_hbm, edata_hbm, zeros_hbm, out_hbm,
                 ebufs, cbufs, rowss, acc_sh, sems_e, sems_g, sems_s):
    cid, sid, wid = _worker_id()
    @pl.when(sid < NS - 1)
    def _zinit():
        pltpu.sync_copy(zeros_hbm.at[pl.ds(sid * RPT15, RPT15)],
                        acc_sh.at[pl.ds(sid * RPT15, RPT15)])

    @pl.when(sid == NS - 1)
    def _zinit_last():
        pltpu.sync_copy(zeros_hbm.at[pl.ds((NS - 1) * RPT15, N - (NS - 1) * RPT15)],
                        acc_sh.at[pl.ds((NS - 1) * RPT15, N - (NS - 1) * RPT15)])

    plsc.subcore_barrier()

    def scale_rows(rows, ebuf):
        for g in range(CH // 16):
            nv = lax.bitcast_convert_type(ebuf[2, pl.ds(g * 16, 16)],
                                          jnp.float32)
            for j in range(16):
                e = g * 16 + j
                spl = lax.gather(
                    nv, jnp.full((16, 1), j, jnp.int32),
                    lax.GatherDimensionNumbers(
                        offset_dims=(), collapsed_slice_dims=(0,),
                        start_index_map=(0,)),
                    slice_sizes=(1,),
                    mode=lax.GatherScatterMode.PROMISE_IN_BOUNDS)
                for s in range(D // 16):
                    sl = pl.ds(s * 16, 16)
                    rows[e, sl] = rows[e, sl] * spl

    def copy_cidx(b):
        for s in range(CH // 16):
            sl = pl.ds(s * 16, 16)
            cbufs[b][sl] = ebufs[b][1, sl]

    def drain_scatter(b):
        pltpu.make_async_copy(rowss[b], acc_sh.at[cbufs[b]],
                              sems_s[b]).wait()

    for b in range(NB):
        pltpu.async_copy(edata_hbm.at[b * NW + wid], ebufs[b], sems_e[b])

    @pl.loop(0, NOUT)
    def _outer(g):
        dg = []
        for b in range(NB):
            @pl.when(g > 0)
            def _drain():
                drain_scatter(b)
            pltpu.make_async_copy(edata_hbm.at[0], ebufs[b], sems_e[b]).wait()
            dg.append(pltpu.async_copy(x_hbm.at[ebufs[b].at[0]], rowss[b],
                                       sems_g[b]))
        for b in range(NB):
            dg[b].wait()
            copy_cidx(b)
            scale_rows(rowss[b], ebufs[b])
            pltpu.async_copy(rowss[b], acc_sh.at[cbufs[b]], sems_s[b],
                             add=True)

            @pl.when(g < NOUT - 1)
            def _prefetch():
                pltpu.async_copy(
                    edata_hbm.at[((g + 1) * NB + b) * NW + wid],
                    ebufs[b], sems_e[b])

    for b in range(NB):
        drain_scatter(b)

    @pl.when(wid < NEXTRA)
    def _extra():
        c = NFULLR * NW + wid
        pltpu.async_copy(edata_hbm.at[c], ebufs[0], sems_e[0]).wait()
        pltpu.async_copy(x_hbm.at[ebufs[0].at[0]], rowss[0], sems_g[0]).wait()
        copy_cidx(0)
        scale_rows(rowss[0], ebufs[0])
        pltpu.async_copy(rowss[0], acc_sh.at[cbufs[0]], sems_s[0],
                         add=True).wait()

    plsc.subcore_barrier()

    @pl.when(sid < NS - 1)
    def _wb():
        pltpu.sync_copy(acc_sh.at[pl.ds(sid * RPT15, RPT15)],
                        out_hbm.at[cid, pl.ds(sid * RPT15, RPT15)])

    @pl.when(sid == NS - 1)
    def _wb_last():
        pltpu.sync_copy(acc_sh.at[pl.ds((NS - 1) * RPT15, N - (NS - 1) * RPT15)],
                        out_hbm.at[cid, pl.ds((NS - 1) * RPT15, N - (NS - 1) * RPT15)])


def _combine_body(p_ref, out_ref):
    out_ref[...] = p_ref[0] + p_ref[1]


def _layer_body(h_ref, s1_ref, p2_ref, w0_ref, w1_ref, w2_ref, b_ref, out_ref):
    w0 = w0_ref[...] - w2_ref[...]
    w2 = 2.0 * w2_ref[...]
    t2 = p2_ref[0] + p2_ref[1]
    acc = jnp.dot(h_ref[...], w0, preferred_element_type=jnp.float32)
    acc += jnp.dot(s1_ref[...], w1_ref[...], preferred_element_type=jnp.float32)
    acc += jnp.dot(t2, w2, preferred_element_type=jnp.float32)
    acc += b_ref[...]
    out_ref[...] = 1.0 / (1.0 + jnp.exp(-acc))


_RB = 1024

_combine = pl.pallas_call(
    _combine_body,
    grid=(NP // _RB,),
    in_specs=[pl.BlockSpec((NC, _RB, D), lambda i: (0, i, 0))],
    out_specs=pl.BlockSpec((_RB, D), lambda i: (i, 0)),
    out_shape=jax.ShapeDtypeStruct((NP, D), jnp.float32),
)

_layer = pl.pallas_call(
    _layer_body,
    grid=(NP // _RB,),
    in_specs=[
        pl.BlockSpec((_RB, D), lambda i: (i, 0)),
        pl.BlockSpec((_RB, D), lambda i: (i, 0)),
        pl.BlockSpec((NC, _RB, D), lambda i: (0, i, 0)),
        pl.BlockSpec((D, D), lambda i: (0, 0)),
        pl.BlockSpec((D, D), lambda i: (0, 0)),
        pl.BlockSpec((D, D), lambda i: (0, 0)),
        pl.BlockSpec((1, D), lambda i: (0, 0)),
    ],
    out_specs=pl.BlockSpec((_RB, D), lambda i: (i, 0)),
    out_shape=jax.ShapeDtypeStruct((NP, D), jnp.float32),
)


def kernel(x, edge_index, edge_weight, W1, b1, W2, b2, W3, b3):
    row = edge_index[0]
    col = edge_index[1]
    zeros_np = jnp.zeros((NP,), jnp.float32)
    zeros_nd = jnp.zeros((NP, D), jnp.float32)

    edata = _prep_kernel(row, col, edge_weight, zeros_np)

    h = jnp.pad(x, ((0, NP - N), (0, 0)))
    for W, b in ((W1, b1), (W2, b2), (W3, b3)):
        p1 = _spmm_kernel(h, edata, zeros_nd)
        s1 = _combine(p1)
        p2 = _spmm_kernel(s1, edata, zeros_nd)
        h = _layer(h, s1, p2, W[0], W[1], W[2], b.reshape(1, D))
    return h[:N]


# speedup vs baseline: 1.4444x; 1.3708x over previous
import functools

import jax
import jax.numpy as jnp
from jax import lax
from jax.experimental import pallas as pl
from jax.experimental.pallas import tpu as pltpu
from jax.experimental.pallas import tpu_sc as plsc

N = 10000
NP = 10240
E = 320000
D = 128
NC = 2
NS = 16
NW = NC * NS
EPW = E // NW
CH = 128
CHD = 128
RPT = N // NS
PPT = NP // NS

_MESH = plsc.VectorSubcoreMesh(core_axis_name="c", subcore_axis_name="s")


def _worker_id():
    cid = lax.axis_index("c")
    sid = lax.axis_index("s")
    return cid, sid, sid * NC + cid


NCHUNK = E // CH
NFULLR = NCHUNK // NW
NEXTRA = NCHUNK - NFULLR * NW
NCHUNKD = E // CHD
NFULLT = NCHUNKD // NS
NEXTRT = NCHUNKD - NFULLT * NS


def _rsqrt16(x):
    xi = lax.bitcast_convert_type(x, jnp.int32)
    yi = jnp.full((16,), 0x5F3759DF, jnp.int32) - lax.shift_right_logical(
        xi, jnp.full((16,), 1, jnp.int32))
    y = lax.bitcast_convert_type(yi, jnp.float32)
    for _ in range(3):
        y = y * (1.5 - 0.5 * x * y * y)
    return y


@functools.partial(
    pl.kernel,
    out_type=jax.ShapeDtypeStruct((NCHUNK, 3, CH), jnp.int32),
    mesh=_MESH,
    scratch_types=[
        pltpu.VMEM((3, CH), jnp.int32),
        pltpu.VMEM((CH,), jnp.float32),
        pltpu.VMEM((CH,), jnp.float32),
        pltpu.VMEM((CH,), jnp.float32),
        pltpu.VMEM((CHD,), jnp.int32),
        pltpu.VMEM((CHD,), jnp.float32),
        pltpu.VMEM((PPT,), jnp.float32),
        pltpu.VMEM_SHARED((NP,), jnp.float32),
        pltpu.VMEM_SHARED((NP,), jnp.float32),
        pltpu.SemaphoreType.DMA,
        pltpu.SemaphoreType.DMA,
    ],
)
def _prep_kernel(row_hbm, col_hbm, w_hbm, zeros_hbm, out_hbm,
                 ebuf, w_v, dr_v, dc_v, didx_v, dw_v, dbuf,
                 deg_sh, dinv_sh, sem_r, sem_c):
    _, sid, wid = _worker_id()

    pltpu.sync_copy(zeros_hbm.at[pl.ds(sid * PPT, PPT)],
                    deg_sh.at[pl.ds(sid * PPT, PPT)])
    plsc.subcore_barrier()

    def deg_chunk(c):
        off = pl.multiple_of(c * CHD, 8)
        pltpu.sync_copy(row_hbm.at[pl.ds(off, CHD)], didx_v)
        pltpu.sync_copy(w_hbm.at[pl.ds(off, CHD)], dw_v)
        pltpu.sync_copy(dw_v, deg_sh.at[didx_v], add=True)

    @pl.loop(0, NFULLT)
    def _deg(g):
        deg_chunk(g * NS + sid)

    @pl.when(sid < NEXTRT)
    def _deg_extra():
        deg_chunk(NFULLT * NS + sid)

    plsc.subcore_barrier()

    pltpu.sync_copy(deg_sh.at[pl.ds(sid * PPT, PPT)], dbuf)
    for j in range(PPT // 16):
        sl = pl.ds(j * 16, 16)
        d = dbuf[sl]
        pos = d > 0.0
        safe = jnp.where(pos, d, 1.0)
        dbuf[sl] = jnp.where(pos, _rsqrt16(safe), 0.0)
    pltpu.sync_copy(dbuf, dinv_sh.at[pl.ds(sid * PPT, PPT)])
    plsc.subcore_barrier()

    def norm_chunk(c):
        off = pl.multiple_of(c * CH, 8)
        pltpu.sync_copy(row_hbm.at[pl.ds(off, CH)], ebuf.at[0])
        pltpu.sync_copy(col_hbm.at[pl.ds(off, CH)], ebuf.at[1])
        pltpu.sync_copy(w_hbm.at[pl.ds(off, CH)], w_v)
        a = pltpu.async_copy(dinv_sh.at[ebuf.at[0]], dr_v, sem_r)
        b = pltpu.async_copy(dinv_sh.at[ebuf.at[1]], dc_v, sem_c)
        a.wait()
        b.wait()
        for j in range(CH // 16):
            sl = pl.ds(j * 16, 16)
            ebuf[2, sl] = lax.bitcast_convert_type(
                -(dr_v[sl] * w_v[sl] * dc_v[sl]), jnp.int32)
        pltpu.sync_copy(ebuf, out_hbm.at[c])

    @pl.loop(0, NFULLR)
    def _chunks(g):
        norm_chunk(g * NW + wid)

    @pl.when(wid < NEXTRA)
    def _extra():
        norm_chunk(NFULLR * NW + wid)


NB = 2
NOUT = NFULLR // NB
RPT15 = 632


@functools.partial(
    pl.kernel,
    out_type=jax.ShapeDtypeStruct((NC, NP, D), jnp.float32),
    mesh=_MESH,
    scratch_types=[
        [pltpu.VMEM((3, CH), jnp.int32) for _ in range(NB)],
        [pltpu.VMEM((CH,), jnp.int32) for _ in range(NB)],
        [pltpu.VMEM((CH, D), jnp.float32) for _ in range(NB)],
        pltpu.VMEM_SHARED((N, D), jnp.float32),
        [pltpu.SemaphoreType.DMA for _ in range(NB)],
        [pltpu.SemaphoreType.DMA for _ in range(NB)],
        [pltpu.SemaphoreType.DMA for _ in range(NB)],
    ],
)
def _spmm_kernel(x_hbm, edata_hbm, zeros_hbm, out_hbm,
                 ebufs, cbufs, rowss, acc_sh, sems_e, sems_g, sems_s):
    cid, sid, wid = _worker_id()
    @pl.when(sid < NS - 1)
    def _zinit():
        pltpu.sync_copy(zeros_hbm.at[pl.ds(sid * RPT15, RPT15)],
                        acc_sh.at[pl.ds(sid * RPT15, RPT15)])

    @pl.when(sid == NS - 1)
    def _zinit_last():
        pltpu.sync_copy(zeros_hbm.at[pl.ds((NS - 1) * RPT15, N - (NS - 1) * RPT15)],
                        acc_sh.at[pl.ds((NS - 1) * RPT15, N - (NS - 1) * RPT15)])

    plsc.subcore_barrier()

    def scale_rows(rows, ebuf):
        @pl.loop(0, CH // 16)
        def _groups(g):
            nv = lax.bitcast_convert_type(ebuf[2, pl.ds(g * 16, 16)],
                                          jnp.float32)
            for j in range(16):
                e = g * 16 + j
                spl = lax.gather(
                    nv, jnp.full((16, 1), j, jnp.int32),
                    lax.GatherDimensionNumbers(
                        offset_dims=(), collapsed_slice_dims=(0,),
                        start_index_map=(0,)),
                    slice_sizes=(1,),
                    mode=lax.GatherScatterMode.PROMISE_IN_BOUNDS)
                for s in range(D // 16):
                    sl = pl.ds(s * 16, 16)
                    rows[e, sl] = rows[e, sl] * spl

    def copy_cidx(b):
        for s in range(CH // 16):
            sl = pl.ds(s * 16, 16)
            cbufs[b][sl] = ebufs[b][1, sl]

    def drain_scatter(b):
        pltpu.make_async_copy(rowss[b], acc_sh.at[cbufs[b]],
                              sems_s[b]).wait()

    for b in range(NB):
        pltpu.async_copy(edata_hbm.at[b * NW + wid], ebufs[b], sems_e[b])

    @pl.loop(0, NOUT)
    def _outer(g):
        dg = []
        for b in range(NB):
            @pl.when(g > 0)
            def _drain():
                drain_scatter(b)
            pltpu.make_async_copy(edata_hbm.at[0], ebufs[b], sems_e[b]).wait()
            dg.append(pltpu.async_copy(x_hbm.at[ebufs[b].at[0]], rowss[b],
                                       sems_g[b]))
        for b in range(NB):
            dg[b].wait()
            copy_cidx(b)
            scale_rows(rowss[b], ebufs[b])
            pltpu.async_copy(rowss[b], acc_sh.at[cbufs[b]], sems_s[b],
                             add=True)

            @pl.when(g < NOUT - 1)
            def _prefetch():
                pltpu.async_copy(
                    edata_hbm.at[((g + 1) * NB + b) * NW + wid],
                    ebufs[b], sems_e[b])

    for b in range(NB):
        drain_scatter(b)

    @pl.when(wid < NEXTRA)
    def _extra():
        c = NFULLR * NW + wid
        pltpu.async_copy(edata_hbm.at[c], ebufs[0], sems_e[0]).wait()
        pltpu.async_copy(x_hbm.at[ebufs[0].at[0]], rowss[0], sems_g[0]).wait()
        copy_cidx(0)
        scale_rows(rowss[0], ebufs[0])
        pltpu.async_copy(rowss[0], acc_sh.at[cbufs[0]], sems_s[0],
                         add=True).wait()

    plsc.subcore_barrier()

    @pl.when(sid < NS - 1)
    def _wb():
        pltpu.sync_copy(acc_sh.at[pl.ds(sid * RPT15, RPT15)],
                        out_hbm.at[cid, pl.ds(sid * RPT15, RPT15)])

    @pl.when(sid == NS - 1)
    def _wb_last():
        pltpu.sync_copy(acc_sh.at[pl.ds((NS - 1) * RPT15, N - (NS - 1) * RPT15)],
                        out_hbm.at[cid, pl.ds((NS - 1) * RPT15, N - (NS - 1) * RPT15)])


def _combine_body(p_ref, out_ref):
    out_ref[...] = p_ref[0] + p_ref[1]


def _layer_body(h_ref, s1_ref, p2_ref, w0_ref, w1_ref, w2_ref, b_ref, out_ref):
    w0 = w0_ref[...] - w2_ref[...]
    w2 = 2.0 * w2_ref[...]
    t2 = p2_ref[0] + p2_ref[1]
    acc = jnp.dot(h_ref[...], w0, preferred_element_type=jnp.float32)
    acc += jnp.dot(s1_ref[...], w1_ref[...], preferred_element_type=jnp.float32)
    acc += jnp.dot(t2, w2, preferred_element_type=jnp.float32)
    acc += b_ref[...]
    out_ref[...] = 1.0 / (1.0 + jnp.exp(-acc))


_RB = 1024

_combine = pl.pallas_call(
    _combine_body,
    grid=(NP // _RB,),
    in_specs=[pl.BlockSpec((NC, _RB, D), lambda i: (0, i, 0))],
    out_specs=pl.BlockSpec((_RB, D), lambda i: (i, 0)),
    out_shape=jax.ShapeDtypeStruct((NP, D), jnp.float32),
)

_layer = pl.pallas_call(
    _layer_body,
    grid=(NP // _RB,),
    in_specs=[
        pl.BlockSpec((_RB, D), lambda i: (i, 0)),
        pl.BlockSpec((_RB, D), lambda i: (i, 0)),
        pl.BlockSpec((NC, _RB, D), lambda i: (0, i, 0)),
        pl.BlockSpec((D, D), lambda i: (0, 0)),
        pl.BlockSpec((D, D), lambda i: (0, 0)),
        pl.BlockSpec((D, D), lambda i: (0, 0)),
        pl.BlockSpec((1, D), lambda i: (0, 0)),
    ],
    out_specs=pl.BlockSpec((_RB, D), lambda i: (i, 0)),
    out_shape=jax.ShapeDtypeStruct((NP, D), jnp.float32),
)


def kernel(x, edge_index, edge_weight, W1, b1, W2, b2, W3, b3):
    row = edge_index[0]
    col = edge_index[1]
    zeros_np = jnp.zeros((NP,), jnp.float32)
    zeros_nd = jnp.zeros((NP, D), jnp.float32)

    edata = _prep_kernel(row, col, edge_weight, zeros_np)

    h = jnp.pad(x, ((0, NP - N), (0, 0)))
    for W, b in ((W1, b1), (W2, b2), (W3, b3)):
        p1 = _spmm_kernel(h, edata, zeros_nd)
        s1 = _combine(p1)
        p2 = _spmm_kernel(s1, edata, zeros_nd)
        h = _layer(h, s1, p2, W[0], W[1], W[2], b.reshape(1, D))
    return h[:N]


# speedup vs baseline: 1.5348x; 1.0626x over previous
import functools

import jax
import jax.numpy as jnp
from jax import lax
from jax.experimental import pallas as pl
from jax.experimental.pallas import tpu as pltpu
from jax.experimental.pallas import tpu_sc as plsc

N = 10000
NP = 10240
E = 320000
D = 128
NC = 2
NS = 16
NW = NC * NS
EPW = E // NW
CH = 128
CHD = 128
RPT = N // NS
PPT = NP // NS

_MESH = plsc.VectorSubcoreMesh(core_axis_name="c", subcore_axis_name="s")


def _worker_id():
    cid = lax.axis_index("c")
    sid = lax.axis_index("s")
    return cid, sid, sid * NC + cid


NCHUNK = E // CH
NFULLR = NCHUNK // NW
NEXTRA = NCHUNK - NFULLR * NW
NCHUNKD = E // CHD
NFULLT = NCHUNKD // NS
NEXTRT = NCHUNKD - NFULLT * NS


def _rsqrt16(x):
    xi = lax.bitcast_convert_type(x, jnp.int32)
    yi = jnp.full((16,), 0x5F3759DF, jnp.int32) - lax.shift_right_logical(
        xi, jnp.full((16,), 1, jnp.int32))
    y = lax.bitcast_convert_type(yi, jnp.float32)
    for _ in range(3):
        y = y * (1.5 - 0.5 * x * y * y)
    return y


@functools.partial(
    pl.kernel,
    out_type=jax.ShapeDtypeStruct((NCHUNK, 3, CH), jnp.int32),
    mesh=_MESH,
    scratch_types=[
        pltpu.VMEM((3, CH), jnp.int32),
        pltpu.VMEM((CH,), jnp.float32),
        pltpu.VMEM((CH,), jnp.float32),
        pltpu.VMEM((CH,), jnp.float32),
        pltpu.VMEM((CHD,), jnp.int32),
        pltpu.VMEM((CHD,), jnp.float32),
        pltpu.VMEM((PPT,), jnp.float32),
        pltpu.VMEM_SHARED((NP,), jnp.float32),
        pltpu.VMEM_SHARED((NP,), jnp.float32),
        pltpu.SemaphoreType.DMA,
        pltpu.SemaphoreType.DMA,
    ],
)
def _prep_kernel(row_hbm, col_hbm, w_hbm, zeros_hbm, out_hbm,
                 ebuf, w_v, dr_v, dc_v, didx_v, dw_v, dbuf,
                 deg_sh, dinv_sh, sem_r, sem_c):
    _, sid, wid = _worker_id()

    pltpu.sync_copy(zeros_hbm.at[pl.ds(sid * PPT, PPT)],
                    deg_sh.at[pl.ds(sid * PPT, PPT)])
    plsc.subcore_barrier()

    def deg_chunk(c):
        off = pl.multiple_of(c * CHD, 8)
        pltpu.sync_copy(row_hbm.at[pl.ds(off, CHD)], didx_v)
        pltpu.sync_copy(w_hbm.at[pl.ds(off, CHD)], dw_v)
        pltpu.sync_copy(dw_v, deg_sh.at[didx_v], add=True)

    @pl.loop(0, NFULLT)
    def _deg(g):
        deg_chunk(g * NS + sid)

    @pl.when(sid < NEXTRT)
    def _deg_extra():
        deg_chunk(NFULLT * NS + sid)

    plsc.subcore_barrier()

    pltpu.sync_copy(deg_sh.at[pl.ds(sid * PPT, PPT)], dbuf)
    for j in range(PPT // 16):
        sl = pl.ds(j * 16, 16)
        d = dbuf[sl]
        pos = d > 0.0
        safe = jnp.where(pos, d, 1.0)
        dbuf[sl] = jnp.where(pos, _rsqrt16(safe), 0.0)
    pltpu.sync_copy(dbuf, dinv_sh.at[pl.ds(sid * PPT, PPT)])
    plsc.subcore_barrier()

    def norm_chunk(c):
        off = pl.multiple_of(c * CH, 8)
        pltpu.sync_copy(row_hbm.at[pl.ds(off, CH)], ebuf.at[0])
        pltpu.sync_copy(col_hbm.at[pl.ds(off, CH)], ebuf.at[1])
        pltpu.sync_copy(w_hbm.at[pl.ds(off, CH)], w_v)
        a = pltpu.async_copy(dinv_sh.at[ebuf.at[0]], dr_v, sem_r)
        b = pltpu.async_copy(dinv_sh.at[ebuf.at[1]], dc_v, sem_c)
        a.wait()
        b.wait()
        for j in range(CH // 16):
            sl = pl.ds(j * 16, 16)
            ebuf[2, sl] = lax.bitcast_convert_type(
                -(dr_v[sl] * w_v[sl] * dc_v[sl]), jnp.int32)
        pltpu.sync_copy(ebuf, out_hbm.at[c])

    @pl.loop(0, NFULLR)
    def _chunks(g):
        norm_chunk(g * NW + wid)

    @pl.when(wid < NEXTRA)
    def _extra():
        norm_chunk(NFULLR * NW + wid)


NB = 3
NOUT = NFULLR // NB
RPT15 = 632


@functools.partial(
    pl.kernel,
    out_type=jax.ShapeDtypeStruct((NC, NP, D), jnp.float32),
    mesh=_MESH,
    scratch_types=[
        [pltpu.VMEM((3, CH), jnp.int32) for _ in range(NB)],
        [pltpu.VMEM((CH,), jnp.int32) for _ in range(NB)],
        [pltpu.VMEM((CH, D), jnp.float32) for _ in range(NB)],
        pltpu.VMEM_SHARED((N, D), jnp.float32),
        [pltpu.SemaphoreType.DMA for _ in range(NB)],
        [pltpu.SemaphoreType.DMA for _ in range(NB)],
        [pltpu.SemaphoreType.DMA for _ in range(NB)],
    ],
)
def _spmm_kernel(x_hbm, edata_hbm, zeros_hbm, out_hbm,
                 ebufs, cbufs, rowss, acc_sh, sems_e, sems_g, sems_s):
    cid, sid, wid = _worker_id()
    @pl.when(sid < NS - 1)
    def _zinit():
        pltpu.sync_copy(zeros_hbm.at[pl.ds(sid * RPT15, RPT15)],
                        acc_sh.at[pl.ds(sid * RPT15, RPT15)])

    @pl.when(sid == NS - 1)
    def _zinit_last():
        pltpu.sync_copy(zeros_hbm.at[pl.ds((NS - 1) * RPT15, N - (NS - 1) * RPT15)],
                        acc_sh.at[pl.ds((NS - 1) * RPT15, N - (NS - 1) * RPT15)])

    plsc.subcore_barrier()

    def scale_rows(rows, ebuf):
        @pl.loop(0, CH // 16)
        def _groups(g):
            nv = lax.bitcast_convert_type(ebuf[2, pl.ds(g * 16, 16)],
                                          jnp.float32)
            for j in range(16):
                e = g * 16 + j
                spl = lax.gather(
                    nv, jnp.full((16, 1), j, jnp.int32),
                    lax.GatherDimensionNumbers(
                        offset_dims=(), collapsed_slice_dims=(0,),
                        start_index_map=(0,)),
                    slice_sizes=(1,),
                    mode=lax.GatherScatterMode.PROMISE_IN_BOUNDS)
                for s in range(D // 16):
                    sl = pl.ds(s * 16, 16)
                    rows[e, sl] = rows[e, sl] * spl

    def copy_cidx(b):
        for s in range(CH // 16):
            sl = pl.ds(s * 16, 16)
            cbufs[b][sl] = ebufs[b][1, sl]

    def drain_scatter(b):
        pltpu.make_async_copy(rowss[b], acc_sh.at[cbufs[b]],
                              sems_s[b]).wait()

    for b in range(NB):
        pltpu.async_copy(edata_hbm.at[b * NW + wid], ebufs[b], sems_e[b])

    @pl.loop(0, NOUT)
    def _outer(g):
        dg = []
        for b in range(NB):
            @pl.when(g > 0)
            def _drain():
                drain_scatter(b)
            pltpu.make_async_copy(edata_hbm.at[0], ebufs[b], sems_e[b]).wait()
            dg.append(pltpu.async_copy(x_hbm.at[ebufs[b].at[0]], rowss[b],
                                       sems_g[b]))
        for b in range(NB):
            dg[b].wait()
            copy_cidx(b)
            scale_rows(rowss[b], ebufs[b])
            pltpu.async_copy(rowss[b], acc_sh.at[cbufs[b]], sems_s[b],
                             add=True)

            @pl.when(g < NOUT - 1)
            def _prefetch():
                pltpu.async_copy(
                    edata_hbm.at[((g + 1) * NB + b) * NW + wid],
                    ebufs[b], sems_e[b])

    for b in range(NB):
        drain_scatter(b)

    @pl.when(wid < NEXTRA)
    def _extra():
        c = NFULLR * NW + wid
        pltpu.async_copy(edata_hbm.at[c], ebufs[0], sems_e[0]).wait()
        pltpu.async_copy(x_hbm.at[ebufs[0].at[0]], rowss[0], sems_g[0]).wait()
        copy_cidx(0)
        scale_rows(rowss[0], ebufs[0])
        pltpu.async_copy(rowss[0], acc_sh.at[cbufs[0]], sems_s[0],
                         add=True).wait()

    plsc.subcore_barrier()

    @pl.when(sid < NS - 1)
    def _wb():
        pltpu.sync_copy(acc_sh.at[pl.ds(sid * RPT15, RPT15)],
                        out_hbm.at[cid, pl.ds(sid * RPT15, RPT15)])

    @pl.when(sid == NS - 1)
    def _wb_last():
        pltpu.sync_copy(acc_sh.at[pl.ds((NS - 1) * RPT15, N - (NS - 1) * RPT15)],
                        out_hbm.at[cid, pl.ds((NS - 1) * RPT15, N - (NS - 1) * RPT15)])


def _combine_body(p_ref, out_ref):
    out_ref[...] = p_ref[0] + p_ref[1]


def _layer_body(h_ref, s1_ref, p2_ref, w0_ref, w1_ref, w2_ref, b_ref, out_ref):
    w0 = w0_ref[...] - w2_ref[...]
    w2 = 2.0 * w2_ref[...]
    t2 = p2_ref[0] + p2_ref[1]
    acc = jnp.dot(h_ref[...], w0, preferred_element_type=jnp.float32)
    acc += jnp.dot(s1_ref[...], w1_ref[...], preferred_element_type=jnp.float32)
    acc += jnp.dot(t2, w2, preferred_element_type=jnp.float32)
    acc += b_ref[...]
    out_ref[...] = 1.0 / (1.0 + jnp.exp(-acc))


_RB = 1024

_combine = pl.pallas_call(
    _combine_body,
    grid=(NP // _RB,),
    in_specs=[pl.BlockSpec((NC, _RB, D), lambda i: (0, i, 0))],
    out_specs=pl.BlockSpec((_RB, D), lambda i: (i, 0)),
    out_shape=jax.ShapeDtypeStruct((NP, D), jnp.float32),
)

_layer = pl.pallas_call(
    _layer_body,
    grid=(NP // _RB,),
    in_specs=[
        pl.BlockSpec((_RB, D), lambda i: (i, 0)),
        pl.BlockSpec((_RB, D), lambda i: (i, 0)),
        pl.BlockSpec((NC, _RB, D), lambda i: (0, i, 0)),
        pl.BlockSpec((D, D), lambda i: (0, 0)),
        pl.BlockSpec((D, D), lambda i: (0, 0)),
        pl.BlockSpec((D, D), lambda i: (0, 0)),
        pl.BlockSpec((1, D), lambda i: (0, 0)),
    ],
    out_specs=pl.BlockSpec((_RB, D), lambda i: (i, 0)),
    out_shape=jax.ShapeDtypeStruct((NP, D), jnp.float32),
)


def kernel(x, edge_index, edge_weight, W1, b1, W2, b2, W3, b3):
    row = edge_index[0]
    col = edge_index[1]
    zeros_np = jnp.zeros((NP,), jnp.float32)
    zeros_nd = jnp.zeros((NP, D), jnp.float32)

    edata = _prep_kernel(row, col, edge_weight, zeros_np)

    h = jnp.pad(x, ((0, NP - N), (0, 0)))
    for W, b in ((W1, b1), (W2, b2), (W3, b3)):
        p1 = _spmm_kernel(h, edata, zeros_nd)
        s1 = _combine(p1)
        p2 = _spmm_kernel(s1, edata, zeros_nd)
        h = _layer(h, s1, p2, W[0], W[1], W[2], b.reshape(1, D))
    return h[:N]


# speedup vs baseline: 1.8738x; 1.2209x over previous
import functools

import jax
import jax.numpy as jnp
from jax import lax
from jax.experimental import pallas as pl
from jax.experimental.pallas import tpu as pltpu
from jax.experimental.pallas import tpu_sc as plsc

N = 10000
NP = 10240
E = 320000
D = 128
NC = 2
NS = 16
NW = NC * NS
EPW = E // NW
CH = 128
CHD = 128
RPT = N // NS
PPT = NP // NS

_MESH = plsc.VectorSubcoreMesh(core_axis_name="c", subcore_axis_name="s")


def _worker_id():
    cid = lax.axis_index("c")
    sid = lax.axis_index("s")
    return cid, sid, sid * NC + cid


NCHUNK = E // CH
NFULLR = NCHUNK // NW
NEXTRA = NCHUNK - NFULLR * NW
NCHUNKD = E // CHD
NFULLT = NCHUNKD // NS
NEXTRT = NCHUNKD - NFULLT * NS


def _rsqrt16(x):
    xi = lax.bitcast_convert_type(x, jnp.int32)
    yi = jnp.full((16,), 0x5F3759DF, jnp.int32) - lax.shift_right_logical(
        xi, jnp.full((16,), 1, jnp.int32))
    y = lax.bitcast_convert_type(yi, jnp.float32)
    for _ in range(3):
        y = y * (1.5 - 0.5 * x * y * y)
    return y


@functools.partial(
    pl.kernel,
    out_type=jax.ShapeDtypeStruct((NCHUNK, 3, CH), jnp.int32),
    mesh=_MESH,
    scratch_types=[
        [pltpu.VMEM((3, CH), jnp.int32) for _ in range(2)],
        [pltpu.VMEM((3, CH), jnp.int32) for _ in range(2)],
        [pltpu.VMEM((CH,), jnp.float32) for _ in range(2)],
        [pltpu.VMEM((CH,), jnp.float32) for _ in range(2)],
        [pltpu.VMEM((CH,), jnp.float32) for _ in range(2)],
        [pltpu.VMEM((CHD,), jnp.int32) for _ in range(2)],
        [pltpu.VMEM((CHD,), jnp.float32) for _ in range(2)],
        [pltpu.VMEM((CHD,), jnp.int32) for _ in range(2)],
        [pltpu.VMEM((CHD,), jnp.float32) for _ in range(2)],
        pltpu.VMEM((PPT,), jnp.float32),
        pltpu.VMEM_SHARED((NP,), jnp.float32),
        pltpu.VMEM_SHARED((NP,), jnp.float32),
        [pltpu.SemaphoreType.DMA for _ in range(2)],
        [pltpu.SemaphoreType.DMA for _ in range(2)],
        [pltpu.SemaphoreType.DMA for _ in range(2)],
        [pltpu.SemaphoreType.DMA for _ in range(2)],
    ],
)
def _prep_kernel(row_hbm, col_hbm, w_hbm, zeros_hbm, out_hbm,
                 ebufs, obufs, w_vs, dr_vs, dc_vs, didxs, dws, pidxs, pws,
                 dbuf, deg_sh, dinv_sh, sems_i, sems_r, sems_c, sems_o):
    _, sid, wid = _worker_id()

    pltpu.sync_copy(zeros_hbm.at[pl.ds(sid * PPT, PPT)],
                    deg_sh.at[pl.ds(sid * PPT, PPT)])
    plsc.subcore_barrier()

    def deg_load(c, b):
        off = pl.multiple_of(c * CHD, 8)
        pltpu.async_copy(row_hbm.at[pl.ds(off, CHD)], didxs[b], sems_i[b])
        pltpu.async_copy(w_hbm.at[pl.ds(off, CHD)], dws[b], sems_i[b])

    def deg_wait(b):
        pltpu.make_async_copy(row_hbm.at[pl.ds(0, CHD)], didxs[b],
                              sems_i[b]).wait()
        pltpu.make_async_copy(w_hbm.at[pl.ds(0, CHD)], dws[b],
                              sems_i[b]).wait()

    def deg_privatize(b):
        @pl.loop(0, CHD // 16)
        def _cp(j):
            sl = pl.ds(j * 16, 16)
            pidxs[b][sl] = didxs[b][sl]
            pws[b][sl] = dws[b][sl]

    def deg_drain(b):
        pltpu.make_async_copy(pws[b], deg_sh.at[pidxs[b]], sems_o[b]).wait()

    for b in range(2):
        deg_load(b * NS + sid, b)

    @pl.loop(0, NFULLT // 2)
    def _deg(g):
        for b in range(2):
            @pl.when(g > 0)
            def _dr():
                deg_drain(b)
            deg_wait(b)
            deg_privatize(b)
            pltpu.async_copy(pws[b], deg_sh.at[pidxs[b]], sems_o[b],
                             add=True)

            @pl.when(g < NFULLT // 2 - 1)
            def _pf():
                deg_load(((g + 1) * 2 + b) * NS + sid, b)

    for b in range(2):
        deg_drain(b)

    @pl.when(sid < NEXTRT)
    def _deg_extra():
        c = NFULLT * NS + sid
        deg_load(c, 0)
        deg_wait(0)
        pltpu.sync_copy(dws[0], deg_sh.at[didxs[0]], add=True)

    plsc.subcore_barrier()

    pltpu.sync_copy(deg_sh.at[pl.ds(sid * PPT, PPT)], dbuf)

    @pl.loop(0, PPT // 16)
    def _dinv(j):
        sl = pl.ds(j * 16, 16)
        d = dbuf[sl]
        pos = d > 0.0
        safe = jnp.where(pos, d, 1.0)
        dbuf[sl] = jnp.where(pos, _rsqrt16(safe), 0.0)

    pltpu.sync_copy(dbuf, dinv_sh.at[pl.ds(sid * PPT, PPT)])
    plsc.subcore_barrier()

    def nrm_load(c, b):
        off = pl.multiple_of(c * CH, 8)
        pltpu.async_copy(row_hbm.at[pl.ds(off, CH)], ebufs[b].at[0],
                         sems_i[b])
        pltpu.async_copy(col_hbm.at[pl.ds(off, CH)], ebufs[b].at[1],
                         sems_i[b])
        pltpu.async_copy(w_hbm.at[pl.ds(off, CH)], w_vs[b], sems_i[b])

    def nrm_wait_in(b):
        pltpu.make_async_copy(row_hbm.at[pl.ds(0, CH)], ebufs[b].at[0],
                              sems_i[b]).wait()
        pltpu.make_async_copy(col_hbm.at[pl.ds(0, CH)], ebufs[b].at[1],
                              sems_i[b]).wait()
        pltpu.make_async_copy(w_hbm.at[pl.ds(0, CH)], w_vs[b],
                              sems_i[b]).wait()

    def nrm_out_drain(b):
        pltpu.make_async_copy(obufs[b], out_hbm.at[0], sems_o[b]).wait()

    def nrm_body(c, b):
        dr = pltpu.async_copy(dinv_sh.at[ebufs[b].at[0]], dr_vs[b],
                              sems_r[b])
        dc = pltpu.async_copy(dinv_sh.at[ebufs[b].at[1]], dc_vs[b],
                              sems_c[b])
        dr.wait()
        dc.wait()

        @pl.loop(0, CH // 16)
        def _nrm(j):
            sl = pl.ds(j * 16, 16)
            obufs[b][0, sl] = ebufs[b][0, sl]
            obufs[b][1, sl] = ebufs[b][1, sl]
            obufs[b][2, sl] = lax.bitcast_convert_type(
                -(dr_vs[b][sl] * w_vs[b][sl] * dc_vs[b][sl]), jnp.int32)

        pltpu.async_copy(obufs[b], out_hbm.at[c], sems_o[b])

    for b in range(2):
        nrm_load((b * NW + wid), b)

    @pl.loop(0, NFULLR // 2)
    def _chunks(g):
        for b in range(2):
            @pl.when(g > 0)
            def _dr():
                nrm_out_drain(b)
            nrm_wait_in(b)
            nrm_body((g * 2 + b) * NW + wid, b)

            @pl.when(g < NFULLR // 2 - 1)
            def _pf():
                nrm_load(((g + 1) * 2 + b) * NW + wid, b)

    for b in range(2):
        nrm_out_drain(b)

    @pl.when(wid < NEXTRA)
    def _extra():
        c = NFULLR * NW + wid
        nrm_load(c, 0)
        nrm_wait_in(0)
        nrm_body(c, 0)
        nrm_out_drain(0)


NB = 3
NOUT = NFULLR // NB
RPT15 = 632


@functools.partial(
    pl.kernel,
    out_type=jax.ShapeDtypeStruct((NC, NP, D), jnp.float32),
    mesh=_MESH,
    scratch_types=[
        [pltpu.VMEM((3, CH), jnp.int32) for _ in range(NB)],
        [pltpu.VMEM((CH,), jnp.int32) for _ in range(NB)],
        [pltpu.VMEM((CH, D), jnp.float32) for _ in range(NB)],
        pltpu.VMEM_SHARED((N, D), jnp.float32),
        [pltpu.SemaphoreType.DMA for _ in range(NB)],
        [pltpu.SemaphoreType.DMA for _ in range(NB)],
        [pltpu.SemaphoreType.DMA for _ in range(NB)],
    ],
)
def _spmm_kernel(x_hbm, edata_hbm, zeros_hbm, out_hbm,
                 ebufs, cbufs, rowss, acc_sh, sems_e, sems_g, sems_s):
    cid, sid, wid = _worker_id()
    @pl.when(sid < NS - 1)
    def _zinit():
        pltpu.sync_copy(zeros_hbm.at[pl.ds(sid * RPT15, RPT15)],
                        acc_sh.at[pl.ds(sid * RPT15, RPT15)])

    @pl.when(sid == NS - 1)
    def _zinit_last():
        pltpu.sync_copy(zeros_hbm.at[pl.ds((NS - 1) * RPT15, N - (NS - 1) * RPT15)],
                        acc_sh.at[pl.ds((NS - 1) * RPT15, N - (NS - 1) * RPT15)])

    plsc.subcore_barrier()

    def scale_rows(rows, ebuf):
        @pl.loop(0, CH // 16)
        def _groups(g):
            nv = lax.bitcast_convert_type(ebuf[2, pl.ds(g * 16, 16)],
                                          jnp.float32)
            for j in range(16):
                e = g * 16 + j
                spl = lax.gather(
                    nv, jnp.full((16, 1), j, jnp.int32),
                    lax.GatherDimensionNumbers(
                        offset_dims=(), collapsed_slice_dims=(0,),
                        start_index_map=(0,)),
                    slice_sizes=(1,),
                    mode=lax.GatherScatterMode.PROMISE_IN_BOUNDS)
                for s in range(D // 16):
                    sl = pl.ds(s * 16, 16)
                    rows[e, sl] = rows[e, sl] * spl

    def copy_cidx(b):
        for s in range(CH // 16):
            sl = pl.ds(s * 16, 16)
            cbufs[b][sl] = ebufs[b][1, sl]

    def drain_scatter(b):
        pltpu.make_async_copy(rowss[b], acc_sh.at[cbufs[b]],
                              sems_s[b]).wait()

    for b in range(NB):
        pltpu.async_copy(edata_hbm.at[b * NW + wid], ebufs[b], sems_e[b])

    @pl.loop(0, NOUT)
    def _outer(g):
        dg = []
        for b in range(NB):
            @pl.when(g > 0)
            def _drain():
                drain_scatter(b)
            pltpu.make_async_copy(edata_hbm.at[0], ebufs[b], sems_e[b]).wait()
            dg.append(pltpu.async_copy(x_hbm.at[ebufs[b].at[0]], rowss[b],
                                       sems_g[b]))
        for b in range(NB):
            dg[b].wait()
            copy_cidx(b)
            scale_rows(rowss[b], ebufs[b])
            pltpu.async_copy(rowss[b], acc_sh.at[cbufs[b]], sems_s[b],
                             add=True)

            @pl.when(g < NOUT - 1)
            def _prefetch():
                pltpu.async_copy(
                    edata_hbm.at[((g + 1) * NB + b) * NW + wid],
                    ebufs[b], sems_e[b])

    for b in range(NB):
        drain_scatter(b)

    @pl.when(wid < NEXTRA)
    def _extra():
        c = NFULLR * NW + wid
        pltpu.async_copy(edata_hbm.at[c], ebufs[0], sems_e[0]).wait()
        pltpu.async_copy(x_hbm.at[ebufs[0].at[0]], rowss[0], sems_g[0]).wait()
        copy_cidx(0)
        scale_rows(rowss[0], ebufs[0])
        pltpu.async_copy(rowss[0], acc_sh.at[cbufs[0]], sems_s[0],
                         add=True).wait()

    plsc.subcore_barrier()

    @pl.when(sid < NS - 1)
    def _wb():
        pltpu.sync_copy(acc_sh.at[pl.ds(sid * RPT15, RPT15)],
                        out_hbm.at[cid, pl.ds(sid * RPT15, RPT15)])

    @pl.when(sid == NS - 1)
    def _wb_last():
        pltpu.sync_copy(acc_sh.at[pl.ds((NS - 1) * RPT15, N - (NS - 1) * RPT15)],
                        out_hbm.at[cid, pl.ds((NS - 1) * RPT15, N - (NS - 1) * RPT15)])


def _combine_body(p_ref, out_ref):
    out_ref[...] = p_ref[0] + p_ref[1]


def _layer_body(h_ref, s1_ref, p2_ref, w0_ref, w1_ref, w2_ref, b_ref, out_ref):
    w0 = w0_ref[...] - w2_ref[...]
    w2 = 2.0 * w2_ref[...]
    t2 = p2_ref[0] + p2_ref[1]
    acc = jnp.dot(h_ref[...], w0, preferred_element_type=jnp.float32)
    acc += jnp.dot(s1_ref[...], w1_ref[...], preferred_element_type=jnp.float32)
    acc += jnp.dot(t2, w2, preferred_element_type=jnp.float32)
    acc += b_ref[...]
    out_ref[...] = 1.0 / (1.0 + jnp.exp(-acc))


_RB = 1024

_combine = pl.pallas_call(
    _combine_body,
    grid=(NP // _RB,),
    in_specs=[pl.BlockSpec((NC, _RB, D), lambda i: (0, i, 0))],
    out_specs=pl.BlockSpec((_RB, D), lambda i: (i, 0)),
    out_shape=jax.ShapeDtypeStruct((NP, D), jnp.float32),
)

_layer = pl.pallas_call(
    _layer_body,
    grid=(NP // _RB,),
    in_specs=[
        pl.BlockSpec((_RB, D), lambda i: (i, 0)),
        pl.BlockSpec((_RB, D), lambda i: (i, 0)),
        pl.BlockSpec((NC, _RB, D), lambda i: (0, i, 0)),
        pl.BlockSpec((D, D), lambda i: (0, 0)),
        pl.BlockSpec((D, D), lambda i: (0, 0)),
        pl.BlockSpec((D, D), lambda i: (0, 0)),
        pl.BlockSpec((1, D), lambda i: (0, 0)),
    ],
    out_specs=pl.BlockSpec((_RB, D), lambda i: (i, 0)),
    out_shape=jax.ShapeDtypeStruct((NP, D), jnp.float32),
)


def kernel(x, edge_index, edge_weight, W1, b1, W2, b2, W3, b3):
    row = edge_index[0]
    col = edge_index[1]
    zeros_np = jnp.zeros((NP,), jnp.float32)
    zeros_nd = jnp.zeros((NP, D), jnp.float32)

    edata = _prep_kernel(row, col, edge_weight, zeros_np)

    h = jnp.pad(x, ((0, NP - N), (0, 0)))
    for W, b in ((W1, b1), (W2, b2), (W3, b3)):
        p1 = _spmm_kernel(h, edata, zeros_nd)
        s1 = _combine(p1)
        p2 = _spmm_kernel(s1, edata, zeros_nd)
        h = _layer(h, s1, p2, W[0], W[1], W[2], b.reshape(1, D))
    return h[:N]
